# baseline (speedup 1.0000x reference)
import functools

import jax
import jax.numpy as jnp
from jax import lax
from jax.experimental import pallas as pl
from jax.experimental.pallas import tpu as pltpu
from jax.experimental.pallas import tpu_sc as plsc

_D = 64
_B = 1024
_S = 200
_LANES = 16
_BB = 128
_NBH = _B // _BB
_NSQ = 4
_SPW = _S // _NSQ
_DH = _D // 8


def _sc_embed(xprep, table, pos2d):
    mesh = plsc.VectorSubcoreMesh(core_axis_name="c", subcore_axis_name="s")

    @functools.partial(
        pl.kernel,
        mesh=mesh,
        out_type=jax.ShapeDtypeStruct((_S, _DH, _NBH, 8, _BB), jnp.float32),
        scratch_types=[
            pltpu.VMEM((_S, _D), jnp.float32),
            pltpu.VMEM((_SPW, _BB), jnp.int32),
            [pltpu.VMEM((_BB, _D), jnp.float32)] * 2,
            [pltpu.VMEM((_DH, 8, _BB), jnp.float32)] * 2,
            [pltpu.SemaphoreType.DMA] * 2,
            [pltpu.SemaphoreType.DMA] * 2,
        ],
        compiler_params=pltpu.CompilerParams(
            use_tc_tiling_on_sc=False, needs_layout_passes=False),
    )
    def k(x_hbm, tab_hbm, pos_hbm, out_hbm, pos_v, idx_v, gbufs, obufs,
          gsems, wsems):
        wid = lax.axis_index("s") * 2 + lax.axis_index("c")
        sq = wid // _NBH
        bh = wid % _NBH
        s_base = sq * _SPW

        pltpu.sync_copy(pos_hbm, pos_v)
        pltpu.sync_copy(x_hbm.at[wid], idx_v)

        def gather_start(s_loc, j):
            pltpu.make_async_copy(
                tab_hbm.at[idx_v.at[s_loc]], gbufs[j], gsems[j]).start()

        def gather_wait(j):
            pltpu.make_async_copy(
                tab_hbm.at[idx_v.at[0]], gbufs[j], gsems[j]).wait()

        def write_dst(s_abs):
            return out_hbm.at[s_abs, :, bh, :, :]

        def write_wait(j):
            pltpu.make_async_copy(obufs[j], write_dst(0), wsems[j]).wait()

        gather_start(0, 0)
        gather_start(1, 1)

        iotas = [jnp.arange(16, dtype=jnp.int32) + _LANES * jv
                 for jv in range(_BB // _LANES)]

        def plane(s_loc, j):
            gbuf, obuf = gbufs[j], obufs[j]

            def dbody(d, _):
                dh = d // 8
                dl = d % 8
                sp = jnp.full((16,), s_base + s_loc, jnp.int32)
                dp = jnp.full((16,), d, jnp.int32)
                padd = plsc.load_gather(pos_v, [sp, dp])
                for jv in range(_BB // _LANES):
                    v = plsc.load_gather(gbuf, [iotas[jv], dp])
                    obuf[dh, dl, pl.ds(_LANES * jv, _LANES)] = v + padd
                return 0

            lax.fori_loop(0, _D, dbody, 0, unroll=4)

        def step(t, _):
            for j in range(2):
                s_loc = 2 * t + j
                gather_wait(j)

                @pl.when(t > 0)
                def _():
                    write_wait(j)

                plane(s_loc, j)
                pltpu.make_async_copy(
                    obufs[j], write_dst(s_base + s_loc), wsems[j]).start()

                @pl.when(t < _SPW // 2 - 1)
                def _():
                    gather_start(s_loc + 2, j)

            return 0

        lax.fori_loop(0, _SPW // 2, step, 0)

        write_wait(0)
        write_wait(1)

    return k(xprep, table, pos2d)


def kernel(x, token_emb, pos_emb):
    seq = x.shape[1]
    xprep = (x.T.astype(jnp.int32)
             .reshape(_NSQ, _SPW, _NBH, _BB)
             .transpose(0, 2, 1, 3)
             .reshape(_NSQ * _NBH, _SPW, _BB))
    pos2d = pos_emb[0, :seq, :].astype(jnp.float32)
    out5 = _sc_embed(xprep, token_emb, pos2d)
    return jnp.transpose(out5, (2, 4, 0, 1, 3)).reshape(_B, _S, _D)


# speedup vs baseline: 2.0317x; 2.0317x over previous
import functools

import jax
import jax.numpy as jnp
from jax import lax
from jax.experimental import pallas as pl
from jax.experimental.pallas import tpu as pltpu
from jax.experimental.pallas import tpu_sc as plsc

_D = 64
_B = 1024
_S = 200
_LANES = 16
_BB = 128
_NBH = _B // _BB
_NSQ = 4
_SPW = _S // _NSQ
_DH = _D // 8
_OP = 136


def _sc_embed(xprep, table, pos2d):
    mesh = plsc.VectorSubcoreMesh(core_axis_name="c", subcore_axis_name="s")

    @functools.partial(
        pl.kernel,
        mesh=mesh,
        out_type=jax.ShapeDtypeStruct((_S, _DH, _NBH, 8, _BB), jnp.float32),
        scratch_types=[
            pltpu.VMEM((_S, _D), jnp.float32),
            pltpu.VMEM((_SPW, _BB), jnp.int32),
            [pltpu.VMEM((_BB, _D), jnp.float32)] * 2,
            [pltpu.VMEM((_DH, 8, _OP), jnp.float32)] * 2,
            [pltpu.SemaphoreType.DMA] * 2,
            [pltpu.SemaphoreType.DMA] * 2,
        ],
        compiler_params=pltpu.CompilerParams(
            use_tc_tiling_on_sc=False, needs_layout_passes=False),
    )
    def k(x_hbm, tab_hbm, pos_hbm, out_hbm, pos_v, idx_v, gbufs, obufs,
          gsems, wsems):
        wid = lax.axis_index("s") * 2 + lax.axis_index("c")
        sq = wid // _NBH
        bh = wid % _NBH
        s_base = sq * _SPW

        pltpu.sync_copy(pos_hbm, pos_v)
        pltpu.sync_copy(x_hbm.at[wid], idx_v)

        def gather_start(s_loc, j):
            pltpu.make_async_copy(
                tab_hbm.at[idx_v.at[s_loc]], gbufs[j], gsems[j]).start()

        def gather_wait(j):
            pltpu.make_async_copy(
                tab_hbm.at[idx_v.at[0]], gbufs[j], gsems[j]).wait()

        def write_dst(s_abs):
            return out_hbm.at[s_abs, :, bh, :, :]

        def obuf_src(j):
            return obufs[j].at[:, :, pl.ds(0, _BB)]

        def write_wait(j):
            pltpu.make_async_copy(obuf_src(j), write_dst(0), wsems[j]).wait()

        gather_start(0, 0)
        gather_start(1, 1)

        iota = jnp.arange(16, dtype=jnp.int32)
        didx = [(( _LANES * i + iota) // 8, (_LANES * i + iota) % 8)
                for i in range(_D // _LANES)]

        def plane(s_loc, j):
            gbuf, obuf = gbufs[j], obufs[j]
            s_abs = s_base + s_loc
            prow = [pos_v[s_abs, pl.ds(_LANES * i, _LANES)]
                    for i in range(_D // _LANES)]

            def bbody(bl, _):
                blp = jnp.full((16,), bl, jnp.int32)
                for i in range(_D // _LANES):
                    v = gbuf[bl, pl.ds(_LANES * i, _LANES)] + prow[i]
                    plsc.store_scatter(obuf, [didx[i][0], didx[i][1], blp], v)
                return 0

            lax.fori_loop(0, _BB, bbody, 0, unroll=4)

        def step(t, _):
            for j in range(2):
                s_loc = 2 * t + j
                gather_wait(j)

                @pl.when(t > 0)
                def _():
                    write_wait(j)

                plane(s_loc, j)
                pltpu.make_async_copy(
                    obuf_src(j), write_dst(s_base + s_loc), wsems[j]).start()

                @pl.when(t < _SPW // 2 - 1)
                def _():
                    gather_start(s_loc + 2, j)

            return 0

        lax.fori_loop(0, _SPW // 2, step, 0)

        write_wait(0)
        write_wait(1)

    return k(xprep, table, pos2d)


def kernel(x, token_emb, pos_emb):
    seq = x.shape[1]
    xprep = (x.T.astype(jnp.int32)
             .reshape(_NSQ, _SPW, _NBH, _BB)
             .transpose(0, 2, 1, 3)
             .reshape(_NSQ * _NBH, _SPW, _BB))
    pos2d = pos_emb[0, :seq, :].astype(jnp.float32)
    out5 = _sc_embed(xprep, token_emb, pos2d)
    return jnp.transpose(out5, (2, 4, 0, 1, 3)).reshape(_B, _S, _D)
